# baseline (speedup 1.0000x reference)
import functools

import jax
import jax.numpy as jnp
from jax import lax
from jax.experimental import pallas as pl
from jax.experimental.pallas import tpu as pltpu
from jax.experimental.pallas import tpu_sc as plsc

B = 4
N = 10000
H = 128
E = 320000

NC = 2
NS = 16

EDGES_PER_TILE = E // NS
CHUNK = 125
NCHUNK = EDGES_PER_TILE // CHUNK
BLK = 16
NBLK = NCHUNK // BLK
ROWS_MAIN = 624
ROWS_LAST = N - (NS - 1) * ROWS_MAIN

_sc_mesh = plsc.VectorSubcoreMesh(core_axis_name="c", subcore_axis_name="s")


@functools.partial(
    pl.kernel,
    out_type=jax.ShapeDtypeStruct((B * N, H), jnp.float32),
    mesh=_sc_mesh,
    scratch_types=[
        pltpu.VMEM((BLK, CHUNK), jnp.int32),
        pltpu.VMEM((BLK, CHUNK), jnp.int32),
        pltpu.VMEM((BLK, CHUNK), jnp.int32),
        pltpu.VMEM((BLK, CHUNK), jnp.int32),
        pltpu.VMEM((CHUNK, H), jnp.float32),
        pltpu.VMEM((CHUNK, H), jnp.float32),
        pltpu.VMEM_SHARED((N, H), jnp.float32),
        pltpu.SemaphoreType.DMA,
        pltpu.SemaphoreType.DMA,
        pltpu.SemaphoreType.DMA,
        pltpu.SemaphoreType.DMA,
        pltpu.SemaphoreType.DMA,
        pltpu.SemaphoreType.DMA,
    ],
)
def _sc_segment_sum(emb3, src_t, dst_t, zeros, out,
                    svA, dvA, svB, dvB, gbuf0, gbuf1, acc,
                    sem0, sem1, semA, semB, ssem0, ssem1):
    c = lax.axis_index("c")
    s = lax.axis_index("s")
    row0 = s * ROWS_MAIN
    emb = emb3.at[0]
    gbufs = (gbuf0, gbuf1)
    gsems = (sem0, sem1)
    ssems = (ssem0, ssem1)

    def wait_gather(par):
        pltpu.make_async_copy(emb.at[svA.at[0]], gbufs[par], gsems[par]).wait()

    def wait_scatter(par):
        pltpu.make_async_copy(emb.at[svA.at[0]], gbufs[par], ssems[par]).wait()

    for k in range(B // NC):
        b = NC * c + k
        emb = emb3.at[b]

        @pl.when(s < NS - 1)
        def _():
            pltpu.sync_copy(zeros.at[pl.ds(0, ROWS_MAIN)],
                            acc.at[pl.ds(row0, ROWS_MAIN)])

        @pl.when(s == NS - 1)
        def _():
            pltpu.sync_copy(zeros, acc.at[pl.ds((NS - 1) * ROWS_MAIN, ROWS_LAST)])

        plsc.subcore_barrier()

        pltpu.sync_copy(src_t.at[s].at[pl.ds(0, BLK)], svA)
        pltpu.sync_copy(dst_t.at[s].at[pl.ds(0, BLK)], dvA)
        pltpu.async_copy(emb.at[svA.at[0]], gbuf0, sem0)

        def run_block(sv, dv, sv_next, sem_next, issue_cond, first_wait_cond):
            for j in range(BLK):
                par = j % 2
                wait_gather(par)
                if j == 0 and first_wait_cond is not None:
                    pl.when(first_wait_cond)(lambda: wait_scatter(1 - par))
                else:
                    wait_scatter(1 - par)
                if j + 1 < BLK:
                    pltpu.async_copy(emb.at[sv.at[j + 1]],
                                     gbufs[1 - par], gsems[1 - par])
                else:
                    def issue_next():
                        pltpu.make_async_copy(
                            src_t.at[s].at[pl.ds(0, BLK)],
                            sv_next, sem_next).wait()
                        pltpu.make_async_copy(
                            src_t.at[s].at[pl.ds(0, BLK)],
                            sv_next, sem_next).wait()
                        pltpu.async_copy(emb.at[sv_next.at[0]],
                                         gbufs[1 - par], gsems[1 - par])
                    if issue_cond is None:
                        issue_next()
                    else:
                        pl.when(issue_cond)(issue_next)
                pltpu.async_copy(gbufs[par], acc.at[dv.at[j]], ssems[par],
                                 add=True)

        def pair_body(p, carry):
            iB = 2 * p + 1
            pltpu.async_copy(src_t.at[s].at[pl.ds(iB * BLK, BLK)], svB, semB)
            pltpu.async_copy(dst_t.at[s].at[pl.ds(iB * BLK, BLK)], dvB, semB)
            run_block(svA, dvA, svB, semB, None, p > 0)
            iA = 2 * p + 2

            @pl.when(iA < NBLK)
            def _():
                pltpu.async_copy(src_t.at[s].at[pl.ds(iA * BLK, BLK)], svA, semA)
                pltpu.async_copy(dst_t.at[s].at[pl.ds(iA * BLK, BLK)], dvA, semA)

            run_block(svB, dvB, svA, semA, p < (NBLK // 2) - 1, None)
            return carry

        lax.fori_loop(0, NBLK // 2, pair_body, 0)
        wait_scatter(1)
        plsc.subcore_barrier()

        @pl.when(s < NS - 1)
        def _():
            pltpu.sync_copy(acc.at[pl.ds(row0, ROWS_MAIN)],
                            out.at[pl.ds(b * N + row0, ROWS_MAIN)])

        @pl.when(s == NS - 1)
        def _():
            pltpu.sync_copy(
                acc.at[pl.ds((NS - 1) * ROWS_MAIN, ROWS_LAST)],
                out.at[pl.ds(b * N + (NS - 1) * ROWS_MAIN, ROWS_LAST)])

        plsc.subcore_barrier()


def _mlp_body(nb_ref, x_ref, m_ref, w1a_ref, w1b_ref, b1_ref, w2_ref, b2_ref, out_ref):
    h = jnp.dot(nb_ref[...], w1a_ref[...], preferred_element_type=jnp.float32)
    h += jnp.dot(x_ref[...], w1b_ref[...], preferred_element_type=jnp.float32)
    h = jnp.maximum(h + b1_ref[...], 0.0)
    imp = jnp.dot(h, w2_ref[...], preferred_element_type=jnp.float32) + b2_ref[...]
    out_ref[...] = jnp.where(m_ref[...] != 0, imp, x_ref[...])


MLP_BLK = 2000


def _mlp(nb, x, m, w1a, w1b, b1, w2, b2):
    grid = ((B * N) // MLP_BLK,)
    return pl.pallas_call(
        _mlp_body,
        grid=grid,
        in_specs=[
            pl.BlockSpec((MLP_BLK, H), lambda i: (i, 0)),
            pl.BlockSpec((MLP_BLK, H), lambda i: (i, 0)),
            pl.BlockSpec((MLP_BLK, 1), lambda i: (i, 0)),
            pl.BlockSpec((H, H), lambda i: (0, 0)),
            pl.BlockSpec((H, H), lambda i: (0, 0)),
            pl.BlockSpec((1, H), lambda i: (0, 0)),
            pl.BlockSpec((H, H), lambda i: (0, 0)),
            pl.BlockSpec((1, H), lambda i: (0, 0)),
        ],
        out_specs=pl.BlockSpec((MLP_BLK, H), lambda i: (i, 0)),
        out_shape=jax.ShapeDtypeStruct((B * N, H), jnp.float32),
    )(nb, x, m, w1a, w1b, b1, w2, b2)


@jax.jit
def kernel(node_embeddings, missing_mask, edge_index, W1, b1, W2, b2):
    src = edge_index[0].astype(jnp.int32)
    dst = edge_index[1].astype(jnp.int32)
    emb_flat = node_embeddings.reshape(B * N, H)
    src_t = src.reshape(NS, NCHUNK, CHUNK)
    dst_t = dst.reshape(NS, NCHUNK, CHUNK)
    zeros = jnp.zeros((ROWS_LAST, H), jnp.float32)
    nb_flat = _sc_segment_sum(node_embeddings, src_t, dst_t, zeros)
    mask = missing_mask.reshape(B * N, 1).astype(jnp.int32)
    out_flat = _mlp(nb_flat, emb_flat, mask, W1[:H], W1[H:], b1.reshape(1, H),
                    W2, b2.reshape(1, H))
    return out_flat.reshape(B, N, H)


# speedup vs baseline: 1.0097x; 1.0097x over previous
import functools

import jax
import jax.numpy as jnp
from jax import lax
from jax.experimental import pallas as pl
from jax.experimental.pallas import tpu as pltpu
from jax.experimental.pallas import tpu_sc as plsc

B = 4
N = 10000
H = 128
E = 320000

NC = 2
NS = 16

EDGES_PER_TILE = E // NS
CHUNK = 125
NCHUNK = EDGES_PER_TILE // CHUNK
BLK = 16
NBLK = NCHUNK // BLK
ROWS_MAIN = 624
ROWS_LAST = N - (NS - 1) * ROWS_MAIN

_sc_mesh = plsc.VectorSubcoreMesh(core_axis_name="c", subcore_axis_name="s")


@functools.partial(
    pl.kernel,
    out_type=jax.ShapeDtypeStruct((B * N, H), jnp.float32),
    mesh=_sc_mesh,
    scratch_types=[
        pltpu.VMEM((BLK, CHUNK), jnp.int32),
        pltpu.VMEM((BLK, CHUNK), jnp.int32),
        pltpu.VMEM((BLK, CHUNK), jnp.int32),
        pltpu.VMEM((BLK, CHUNK), jnp.int32),
        pltpu.VMEM((CHUNK, H), jnp.float32),
        pltpu.VMEM((CHUNK, H), jnp.float32),
        pltpu.VMEM_SHARED((N, H), jnp.float32),
        pltpu.SemaphoreType.DMA,
        pltpu.SemaphoreType.DMA,
        pltpu.SemaphoreType.DMA,
        pltpu.SemaphoreType.DMA,
        pltpu.SemaphoreType.DMA,
        pltpu.SemaphoreType.DMA,
    ],
)
def _sc_segment_sum(emb3, src_t, dst_t, zeros, out,
                    svA, dvA, svB, dvB, gbuf0, gbuf1, acc,
                    sem0, sem1, semA, semB, ssem0, ssem1):
    c = lax.axis_index("c")
    s = lax.axis_index("s")
    row0 = s * ROWS_MAIN
    emb = emb3.at[0]
    gbufs = (gbuf0, gbuf1)
    gsems = (sem0, sem1)
    ssems = (ssem0, ssem1)

    def wait_gather(par):
        pltpu.make_async_copy(emb.at[svA.at[0]], gbufs[par], gsems[par]).wait()

    def wait_scatter(par):
        pltpu.make_async_copy(emb.at[svA.at[0]], gbufs[par], ssems[par]).wait()

    for k in range(B // NC):
        b = NC * c + k
        emb = emb3.at[b]

        @pl.when(s < NS - 1)
        def _():
            pltpu.sync_copy(zeros.at[pl.ds(0, ROWS_MAIN)],
                            acc.at[pl.ds(row0, ROWS_MAIN)])

        @pl.when(s == NS - 1)
        def _():
            pltpu.sync_copy(zeros, acc.at[pl.ds((NS - 1) * ROWS_MAIN, ROWS_LAST)])

        plsc.subcore_barrier()

        pltpu.sync_copy(src_t.at[s].at[pl.ds(0, BLK)], svA)
        pltpu.sync_copy(dst_t.at[s].at[pl.ds(0, BLK)], dvA)
        pltpu.async_copy(emb.at[svA.at[0]], gbuf0, sem0)

        def run_block(sv, dv, sv_next, sem_next, issue_cond, first_wait_cond):
            for j in range(BLK):
                par = j % 2
                wait_gather(par)
                if j == 0 and first_wait_cond is not None:
                    pl.when(first_wait_cond)(lambda: wait_scatter(1 - par))
                else:
                    wait_scatter(1 - par)
                if j + 1 < BLK:
                    pltpu.async_copy(emb.at[sv.at[j + 1]],
                                     gbufs[1 - par], gsems[1 - par])
                else:
                    def issue_next():
                        pltpu.make_async_copy(
                            src_t.at[s].at[pl.ds(0, BLK)],
                            sv_next, sem_next).wait()
                        pltpu.make_async_copy(
                            src_t.at[s].at[pl.ds(0, BLK)],
                            sv_next, sem_next).wait()
                        pltpu.async_copy(emb.at[sv_next.at[0]],
                                         gbufs[1 - par], gsems[1 - par])
                    if issue_cond is None:
                        issue_next()
                    else:
                        pl.when(issue_cond)(issue_next)
                pltpu.async_copy(gbufs[par], acc.at[dv.at[j]], ssems[par],
                                 add=True)

        def pair_body(p, carry):
            iB = 2 * p + 1
            pltpu.async_copy(src_t.at[s].at[pl.ds(iB * BLK, BLK)], svB, semB)
            pltpu.async_copy(dst_t.at[s].at[pl.ds(iB * BLK, BLK)], dvB, semB)
            run_block(svA, dvA, svB, semB, None, p > 0)
            iA = 2 * p + 2

            @pl.when(iA < NBLK)
            def _():
                pltpu.async_copy(src_t.at[s].at[pl.ds(iA * BLK, BLK)], svA, semA)
                pltpu.async_copy(dst_t.at[s].at[pl.ds(iA * BLK, BLK)], dvA, semA)

            run_block(svB, dvB, svA, semA, p < (NBLK // 2) - 1, None)
            return carry

        lax.fori_loop(0, NBLK // 2, pair_body, 0)
        wait_scatter(1)
        plsc.subcore_barrier()

        @pl.when(s < NS - 1)
        def _():
            pltpu.sync_copy(acc.at[pl.ds(row0, ROWS_MAIN)],
                            out.at[pl.ds(b * N + row0, ROWS_MAIN)])

        @pl.when(s == NS - 1)
        def _():
            pltpu.sync_copy(
                acc.at[pl.ds((NS - 1) * ROWS_MAIN, ROWS_LAST)],
                out.at[pl.ds(b * N + (NS - 1) * ROWS_MAIN, ROWS_LAST)])

        plsc.subcore_barrier()


def _mlp_body(nb_ref, x_ref, m_ref, w1a_ref, w1b_ref, b1_ref, w2_ref, b2_ref, out_ref):
    h = jnp.dot(nb_ref[...], w1a_ref[...], preferred_element_type=jnp.float32)
    h += jnp.dot(x_ref[...], w1b_ref[...], preferred_element_type=jnp.float32)
    h = jnp.maximum(h + b1_ref[...], 0.0)
    imp = jnp.dot(h, w2_ref[...], preferred_element_type=jnp.float32) + b2_ref[...]
    out_ref[...] = jnp.where(m_ref[...] != 0, imp, x_ref[...])


MLP_BLK = 4000


def _mlp(nb, x, m, w1a, w1b, b1, w2, b2):
    grid = ((B * N) // MLP_BLK,)
    return pl.pallas_call(
        _mlp_body,
        grid=grid,
        in_specs=[
            pl.BlockSpec((MLP_BLK, H), lambda i: (i, 0)),
            pl.BlockSpec((MLP_BLK, H), lambda i: (i, 0)),
            pl.BlockSpec((MLP_BLK, 1), lambda i: (i, 0)),
            pl.BlockSpec((H, H), lambda i: (0, 0)),
            pl.BlockSpec((H, H), lambda i: (0, 0)),
            pl.BlockSpec((1, H), lambda i: (0, 0)),
            pl.BlockSpec((H, H), lambda i: (0, 0)),
            pl.BlockSpec((1, H), lambda i: (0, 0)),
        ],
        out_specs=pl.BlockSpec((MLP_BLK, H), lambda i: (i, 0)),
        out_shape=jax.ShapeDtypeStruct((B * N, H), jnp.float32),
    )(nb, x, m, w1a, w1b, b1, w2, b2)


@jax.jit
def kernel(node_embeddings, missing_mask, edge_index, W1, b1, W2, b2):
    src = edge_index[0].astype(jnp.int32)
    dst = edge_index[1].astype(jnp.int32)
    emb_flat = node_embeddings.reshape(B * N, H)
    src_t = src.reshape(NS, NCHUNK, CHUNK)
    dst_t = dst.reshape(NS, NCHUNK, CHUNK)
    zeros = jnp.zeros((ROWS_LAST, H), jnp.float32)
    nb_flat = _sc_segment_sum(node_embeddings, src_t, dst_t, zeros)
    mask = missing_mask.reshape(B * N, 1).astype(jnp.int32)
    out_flat = _mlp(nb_flat, emb_flat, mask, W1[:H], W1[H:], b1.reshape(1, H),
                    W2, b2.reshape(1, H))
    return out_flat.reshape(B, N, H)
